# Initial kernel scaffold; baseline (speedup 1.0000x reference)
import jax
import jax.numpy as jnp
from jax.experimental import pallas as pl


def kernel(x, edge_index, batch, W1, b1, W2, b2, W_lin, b_lin):
    raise NotImplementedError("write your pallas kernel here")


# speedup vs baseline: 23.9066x; 23.9066x over previous
import functools

import jax
import jax.numpy as jnp
from jax import lax
from jax.experimental import pallas as pl
from jax.experimental.pallas import tpu as pltpu
from jax.experimental.pallas import tpu_sc as plsc

N = 10000
NPAD = 10240
E = 320000
G = 100
GPAD = 128
K = 64
D_IN = 128
D_HID = 64
D2 = 16
SENT = N
NC, NS = 2, 16
NW = NC * NS
EPW = E // NW
CH = 128
NFULL = EPW // CH
REM = EPW - NFULL * CH
RPS = NPAD // NS
RCH = RPS // CH

_mesh = plsc.VectorSubcoreMesh(core_axis_name="c", subcore_axis_name="s",
                               num_cores=NC, num_subcores=NS)
_sc_params = pltpu.CompilerParams(use_tc_tiling_on_sc=False,
                                  needs_layout_passes=False)


def _fill_rows(ref, rows, vec):
  cols = ref.shape[1] // 16

  @pl.loop(0, rows)
  def _(i):
    for c in range(cols):
      ref[i, pl.ds(c * 16, 16)] = vec


def _zero_slice_loop(sid, buf_v, sh_ref):
  @pl.loop(0, RCH)
  def _(j):
    pltpu.sync_copy(buf_v, sh_ref.at[pl.ds(sid * RPS + j * CH, CH)])


def _copy_out_loop(cid, sid, buf_v, sh_ref, out0, out1):
  @pl.loop(0, RCH)
  def _(j):
    off = sid * RPS + j * CH
    pltpu.sync_copy(sh_ref.at[pl.ds(off, CH)], buf_v)

    @pl.when(cid == 0)
    def _():
      pltpu.sync_copy(buf_v, out0.at[pl.ds(off, CH)])

    @pl.when(cid == 1)
    def _():
      pltpu.sync_copy(buf_v, out1.at[pl.ds(off, CH)])


def _sc_deg_body(dst_hbm, deg0, deg1, idx_v, idxr_v, ones_v, buf_v, deg_sh):
  cid = lax.axis_index("c")
  sid = lax.axis_index("s")
  wid = cid * NS + sid
  _fill_rows(ones_v, CH, jnp.full((16,), 1.0, jnp.float32))
  _fill_rows(buf_v, CH, jnp.zeros((16,), jnp.float32))
  _zero_slice_loop(sid, buf_v, deg_sh)
  plsc.subcore_barrier()

  base = wid * EPW

  @pl.loop(0, NFULL)
  def _(i):
    pltpu.sync_copy(dst_hbm.at[pl.ds(base + i * CH, CH)], idx_v)
    pltpu.sync_copy(ones_v, deg_sh.at[idx_v], add=True)

  if REM:
    pltpu.sync_copy(dst_hbm.at[pl.ds(base + NFULL * CH, REM)], idxr_v)
    pltpu.sync_copy(ones_v.at[pl.ds(0, REM)], deg_sh.at[idxr_v], add=True)

  plsc.subcore_barrier()
  _copy_out_loop(cid, sid, buf_v, deg_sh, deg0, deg1)


_sc_deg = pl.kernel(
    _sc_deg_body,
    out_type=(jax.ShapeDtypeStruct((NPAD, 16), jnp.float32),
              jax.ShapeDtypeStruct((NPAD, 16), jnp.float32)),
    mesh=_mesh,
    compiler_params=_sc_params,
    scratch_types=[
        pltpu.VMEM((CH,), jnp.int32),
        pltpu.VMEM((REM,), jnp.int32),
        pltpu.VMEM((CH, 16), jnp.float32),
        pltpu.VMEM((CH, 16), jnp.float32),
        pltpu.VMEM_SHARED((NPAD, 16), jnp.float32),
    ],
)


def _sc_agg_body(src_hbm, dst_hbm, h_hbm, out0, out1,
                 sidx_v, didx_v, sidxr_v, didxr_v, rows_v, buf_v, agg_sh):
  cid = lax.axis_index("c")
  sid = lax.axis_index("s")
  wid = cid * NS + sid
  w = buf_v.shape[1]
  _fill_rows(buf_v, CH, jnp.zeros((16,), jnp.float32))
  _zero_slice_loop(sid, buf_v, agg_sh)
  plsc.subcore_barrier()

  base = wid * EPW

  @pl.loop(0, NFULL)
  def _(i):
    pltpu.sync_copy(src_hbm.at[pl.ds(base + i * CH, CH)], sidx_v)
    pltpu.sync_copy(dst_hbm.at[pl.ds(base + i * CH, CH)], didx_v)
    pltpu.sync_copy(h_hbm.at[sidx_v], rows_v)
    pltpu.sync_copy(rows_v, agg_sh.at[didx_v], add=True)

  if REM:
    off = base + NFULL * CH
    pltpu.sync_copy(src_hbm.at[pl.ds(off, REM)], sidxr_v)
    pltpu.sync_copy(dst_hbm.at[pl.ds(off, REM)], didxr_v)
    pltpu.sync_copy(h_hbm.at[sidxr_v], rows_v.at[pl.ds(0, REM)])
    pltpu.sync_copy(rows_v.at[pl.ds(0, REM)], agg_sh.at[didxr_v], add=True)

  plsc.subcore_barrier()
  _copy_out_loop(cid, sid, buf_v, agg_sh, out0, out1)


def _make_sc_agg(width):
  return pl.kernel(
      _sc_agg_body,
      out_type=(jax.ShapeDtypeStruct((NPAD, width), jnp.float32),
                jax.ShapeDtypeStruct((NPAD, width), jnp.float32)),
      mesh=_mesh,
      compiler_params=_sc_params,
      scratch_types=[
          pltpu.VMEM((CH,), jnp.int32),
          pltpu.VMEM((CH,), jnp.int32),
          pltpu.VMEM((REM,), jnp.int32),
          pltpu.VMEM((REM,), jnp.int32),
          pltpu.VMEM((CH, width), jnp.float32),
          pltpu.VMEM((CH, width), jnp.float32),
          pltpu.VMEM_SHARED((NPAD, width), jnp.float32),
      ],
  )


_sc_agg64 = _make_sc_agg(D_HID)
_sc_agg16 = _make_sc_agg(D2)


def _sc_pool_body(keys_hbm, starts_hbm, counts_hbm, h2_hbm, pooled,
                  keys_v, starts_v, counts_v, sel_v, rows_v):
  cid = lax.axis_index("c")
  sid = lax.axis_index("s")
  wid = cid * NS + sid
  pltpu.sync_copy(keys_hbm, keys_v)
  pltpu.sync_copy(starts_hbm, starts_v)
  pltpu.sync_copy(counts_hbm, counts_v)
  iota16 = lax.iota(jnp.int32, 16)

  for j in range((G + NW - 1) // NW):
    g = wid + NW * j

    @pl.when(g < G)
    def _():
      gsplat = jnp.full((16,), g, jnp.int32)
      start = jnp.max(plsc.load_gather(starts_v, [gsplat]))
      cnt = jnp.max(plsc.load_gather(counts_v, [gsplat]))
      for t in range(K // 16):
        sel_v[pl.ds(t * 16, 16)] = jnp.full((16,), SENT, jnp.int32)

      nq = (cnt + 15) // 16

      def pbody(p, _):
        node = start + p
        keyi = plsc.load_gather(keys_v, [jnp.full((16,), node, jnp.int32)])

        def qbody(q, rv):
          kb = q * 16
          kv = keys_v[pl.ds(start + kb, 16)]
          posv = kb + iota16
          valid = posv < cnt
          beats = jnp.logical_or(kv > keyi,
                                 jnp.logical_and(kv == keyi, posv < p))
          return rv + plsc.all_reduce_population_count(
              jnp.logical_and(valid, beats))

        rv = lax.fori_loop(0, nq, qbody, jnp.zeros((16,), jnp.int32))
        rank = jnp.max(rv)
        ranksplat = jnp.full((16,), rank, jnp.int32)
        msk = jnp.logical_and(iota16 == 0, ranksplat < K)
        plsc.store_scatter(sel_v, [ranksplat],
                           jnp.full((16,), node, jnp.int32), mask=msk)
        return 0

      lax.fori_loop(0, cnt, pbody, 0)
      pltpu.sync_copy(h2_hbm.at[sel_v], rows_v)
      pltpu.sync_copy(rows_v, pooled.at[g])


_sc_pool = pl.kernel(
    _sc_pool_body,
    out_type=jax.ShapeDtypeStruct((G, K, D2), jnp.float32),
    mesh=_mesh,
    compiler_params=_sc_params,
    scratch_types=[
        pltpu.VMEM((NPAD,), jnp.float32),
        pltpu.VMEM((GPAD,), jnp.int32),
        pltpu.VMEM((GPAD,), jnp.int32),
        pltpu.VMEM((K,), jnp.int32),
        pltpu.VMEM((K, D2), jnp.float32),
    ],
)


def _tc_a_body(x_ref, w1_ref, deg0_ref, deg1_ref, hp1_ref):
  deg = deg0_ref[:, 0:1] + deg1_ref[:, 0:1] + 1.0
  dinv = lax.rsqrt(deg)
  h = jnp.dot(x_ref[...], w1_ref[...], preferred_element_type=jnp.float32)
  hp1_ref[...] = h * dinv


def _tc_meta_body(batch_ref, starts_ref, counts_ref):
  b = batch_ref[...]
  gid = lax.broadcasted_iota(jnp.int32, (GPAD, NPAD), 0)
  bmat = jnp.broadcast_to(b, (GPAD, NPAD))
  starts_ref[...] = jnp.sum((bmat < gid).astype(jnp.int32), axis=1,
                            keepdims=True)
  counts_ref[...] = jnp.sum((bmat == gid).astype(jnp.int32), axis=1,
                            keepdims=True)


def _tc_b_body(a0_ref, a1_ref, hp1_ref, deg0_ref, deg1_ref, b1_ref, w2_ref,
               hp2_ref):
  deg = deg0_ref[:, 0:1] + deg1_ref[:, 0:1] + 1.0
  dinv = lax.rsqrt(deg)
  h = jnp.tanh((a0_ref[...] + a1_ref[...] + hp1_ref[...]) * dinv + b1_ref[...])
  hp2_ref[...] = jnp.dot(h, w2_ref[...],
                         preferred_element_type=jnp.float32) * dinv


def _tc_c_body(a0_ref, a1_ref, hp2_ref, deg0_ref, deg1_ref, b2_ref,
               h2_ref, key_ref):
  deg = deg0_ref[:, 0:1] + deg1_ref[:, 0:1] + 1.0
  dinv = lax.rsqrt(deg)
  h2 = jnp.tanh((a0_ref[...] + a1_ref[...] + hp2_ref[...]) * dinv + b2_ref[...])
  rows = lax.broadcasted_iota(jnp.int32, (NPAD, 1), 0)
  h2 = jnp.where(rows < N, h2, 0.0)
  h2_ref[...] = h2
  key_ref[...] = h2[:, 1:2]


def _tc_d_body(flat_ref, wl_ref, bl_ref, out_ref):
  out_ref[...] = jnp.dot(flat_ref[...], wl_ref[...],
                         preferred_element_type=jnp.float32) + bl_ref[...]


def _tc_call(body, out_shapes):
  return pl.pallas_call(body, out_shape=out_shapes)


def kernel(x, edge_index, batch, W1, b1, W2, b2, W_lin, b_lin):
  assert x.shape == (N, D_IN) and edge_index.shape == (2, E)
  src = edge_index[0]
  dst = edge_index[1]

  xp = jnp.pad(x, ((0, NPAD - N), (0, 0)))
  batch_p = jnp.concatenate(
      [batch.astype(jnp.int32), jnp.full((NPAD - N,), 1 << 20, jnp.int32)])

  deg0, deg1 = _sc_deg(dst)

  hp1 = _tc_call(_tc_a_body,
                 jax.ShapeDtypeStruct((NPAD, D_HID), jnp.float32))(
                     xp, W1, deg0, deg1)

  starts, counts = _tc_call(
      _tc_meta_body,
      (jax.ShapeDtypeStruct((GPAD, 1), jnp.int32),
       jax.ShapeDtypeStruct((GPAD, 1), jnp.int32)))(batch_p.reshape(1, NPAD))

  a0, a1 = _sc_agg64(src, dst, hp1)

  w2p = jnp.pad(W2, ((0, 0), (0, D2 - W2.shape[1])))
  b2p = jnp.pad(b2, (0, D2 - b2.shape[0]))
  hp2 = _tc_call(_tc_b_body,
                 jax.ShapeDtypeStruct((NPAD, D2), jnp.float32))(
                     a0, a1, hp1, deg0, deg1, b1.reshape(1, D_HID), w2p)

  c0, c1 = _sc_agg16(src, dst, hp2)

  h2, keys = _tc_call(
      _tc_c_body,
      (jax.ShapeDtypeStruct((NPAD, D2), jnp.float32),
       jax.ShapeDtypeStruct((NPAD, 1), jnp.float32)))(
          c0, c1, hp2, deg0, deg1, b2p.reshape(1, D2))

  pooled = _sc_pool(keys.reshape(NPAD), starts.reshape(GPAD),
                    counts.reshape(GPAD), h2)

  wl = jnp.pad(W_lin.reshape(K, 2, 1), ((0, 0), (0, D2 - 2), (0, 0)))
  out = _tc_call(_tc_d_body, jax.ShapeDtypeStruct((G, 1), jnp.float32))(
      pooled.reshape(G, K * D2), wl.reshape(K * D2, 1),
      b_lin.reshape(1, 1))
  return out


# speedup vs baseline: 47.7612x; 1.9978x over previous
import functools

import jax
import jax.numpy as jnp
from jax import lax
from jax.experimental import pallas as pl
from jax.experimental.pallas import tpu as pltpu
from jax.experimental.pallas import tpu_sc as plsc

N = 10000
NPAD = 10240
E = 320000
G = 100
GPAD = 128
K = 64
D_IN = 128
D_HID = 64
D2 = 16
SENT = N
NC, NS = 2, 16
NW = NC * NS
EPW = E // NW
CH = 128
NFULL = EPW // CH
REM = EPW - NFULL * CH
RPS = NPAD // NS
RCH = RPS // CH

_mesh = plsc.VectorSubcoreMesh(core_axis_name="c", subcore_axis_name="s",
                               num_cores=NC, num_subcores=NS)
_sc_params = pltpu.CompilerParams(use_tc_tiling_on_sc=False,
                                  needs_layout_passes=False)


def _fill_rows(ref, rows, vec):
  cols = ref.shape[1] // 16

  @pl.loop(0, rows)
  def _(i):
    for c in range(cols):
      ref[i, pl.ds(c * 16, 16)] = vec


def _zero_slice_loop(sid, buf_v, sh_ref):
  @pl.loop(0, RCH)
  def _(j):
    pltpu.sync_copy(buf_v, sh_ref.at[pl.ds(sid * RPS + j * CH, CH)])


def _copy_out_loop(cid, sid, buf_v, sh_ref, out0, out1):
  @pl.loop(0, RCH)
  def _(j):
    off = sid * RPS + j * CH
    pltpu.sync_copy(sh_ref.at[pl.ds(off, CH)], buf_v)

    @pl.when(cid == 0)
    def _():
      pltpu.sync_copy(buf_v, out0.at[pl.ds(off, CH)])

    @pl.when(cid == 1)
    def _():
      pltpu.sync_copy(buf_v, out1.at[pl.ds(off, CH)])


RING = 6
assert NFULL % RING == 0
NROUND = NFULL // RING


def _repack_idx(idx1_v, idx2_v):
  @pl.loop(0, (NFULL * CH) // 16)
  def _(t):
    idx2_v[t // (CH // 16), pl.ds((t % (CH // 16)) * 16, 16)] = (
        idx1_v[pl.ds(t * 16, 16)])


def _sc_deg_body(dst_hbm, deg0, deg1, didx1_v, didx2_v, idxr_v, ones_v,
                 buf_v, deg_sh, ssem):
  cid = lax.axis_index("c")
  sid = lax.axis_index("s")
  wid = cid * NS + sid
  _fill_rows(ones_v, CH, jnp.full((16,), 1.0, jnp.float32))
  _fill_rows(buf_v, CH, jnp.zeros((16,), jnp.float32))
  _zero_slice_loop(sid, buf_v, deg_sh)

  base = wid * EPW
  pltpu.sync_copy(dst_hbm.at[pl.ds(base, EPW)], didx1_v)
  _repack_idx(didx1_v, didx2_v)
  if REM:
    idxr_v[...] = didx1_v[pl.ds(NFULL * CH, REM)]
  plsc.subcore_barrier()

  @pl.loop(0, NROUND)
  def _(o):
    ibase = o * RING
    for r in range(RING):
      pltpu.async_copy(ones_v, deg_sh.at[didx2_v.at[ibase + r]], ssem,
                       add=True)
    for r in range(RING):
      pltpu.make_async_copy(ones_v, deg_sh.at[didx2_v.at[ibase + r]],
                            ssem).wait()

  if REM:
    pltpu.sync_copy(ones_v.at[pl.ds(0, REM)], deg_sh.at[idxr_v], add=True)

  plsc.subcore_barrier()
  _copy_out_loop(cid, sid, buf_v, deg_sh, deg0, deg1)


_sc_deg = pl.kernel(
    _sc_deg_body,
    out_type=(jax.ShapeDtypeStruct((NPAD, 16), jnp.float32),
              jax.ShapeDtypeStruct((NPAD, 16), jnp.float32)),
    mesh=_mesh,
    compiler_params=_sc_params,
    scratch_types=[
        pltpu.VMEM((EPW,), jnp.int32),
        pltpu.VMEM((NFULL, CH), jnp.int32),
        pltpu.VMEM((REM,), jnp.int32),
        pltpu.VMEM((CH, 16), jnp.float32),
        pltpu.VMEM((CH, 16), jnp.float32),
        pltpu.VMEM_SHARED((NPAD, 16), jnp.float32),
        pltpu.SemaphoreType.DMA,
    ],
)


def _sc_agg_body(src_hbm, dst_hbm, h_hbm, out0, out1,
                 sidx_v, didx1_v, didx2_v, didxr_v, rows_v, buf_v, agg_sh,
                 gs0, gs1, gs2, gs3, gs4, gs5, ssem):
  cid = lax.axis_index("c")
  sid = lax.axis_index("s")
  wid = cid * NS + sid
  gsems = (gs0, gs1, gs2, gs3, gs4, gs5)
  _fill_rows(buf_v, CH, jnp.zeros((16,), jnp.float32))
  _zero_slice_loop(sid, buf_v, agg_sh)

  base = wid * EPW
  pltpu.sync_copy(src_hbm.at[pl.ds(base, EPW)], sidx_v)
  pltpu.sync_copy(dst_hbm.at[pl.ds(base, EPW)], didx1_v)
  _repack_idx(didx1_v, didx2_v)
  if REM:
    didxr_v[...] = didx1_v[pl.ds(NFULL * CH, REM)]
  plsc.subcore_barrier()

  def g_start(i, r):
    pltpu.async_copy(h_hbm.at[sidx_v.at[pl.ds(i * CH, CH)]], rows_v.at[r],
                     gsems[r])

  def g_wait(i, r):
    pltpu.make_async_copy(h_hbm.at[sidx_v.at[pl.ds(i * CH, CH)]],
                          rows_v.at[r], gsems[r]).wait()

  for r in range(RING):
    g_start(r, r)

  @pl.loop(0, NROUND)
  def _(o):
    ibase = o * RING
    for r in range(RING):
      g_wait(ibase + r, r)
      pltpu.async_copy(rows_v.at[r], agg_sh.at[didx2_v.at[ibase + r]], ssem,
                       add=True)
    for r in range(RING):
      pltpu.make_async_copy(rows_v.at[r], agg_sh.at[didx2_v.at[ibase + r]],
                            ssem).wait()
    for r in range(RING):
      nxt = ibase + RING + r

      @pl.when(nxt < NFULL)
      def _():
        g_start(nxt, r)

  if REM:
    off = NFULL * CH
    pltpu.sync_copy(h_hbm.at[sidx_v.at[pl.ds(off, REM)]],
                    rows_v.at[0, pl.ds(0, REM)])
    pltpu.sync_copy(rows_v.at[0, pl.ds(0, REM)], agg_sh.at[didxr_v],
                    add=True)

  plsc.subcore_barrier()
  _copy_out_loop(cid, sid, buf_v, agg_sh, out0, out1)


def _make_sc_agg(width):
  return pl.kernel(
      _sc_agg_body,
      out_type=(jax.ShapeDtypeStruct((NPAD, width), jnp.float32),
                jax.ShapeDtypeStruct((NPAD, width), jnp.float32)),
      mesh=_mesh,
      compiler_params=_sc_params,
      scratch_types=[
          pltpu.VMEM((EPW,), jnp.int32),
          pltpu.VMEM((EPW,), jnp.int32),
          pltpu.VMEM((NFULL, CH), jnp.int32),
          pltpu.VMEM((REM,), jnp.int32),
          pltpu.VMEM((RING, CH, width), jnp.float32),
          pltpu.VMEM((CH, width), jnp.float32),
          pltpu.VMEM_SHARED((NPAD, width), jnp.float32),
          pltpu.SemaphoreType.DMA,
          pltpu.SemaphoreType.DMA,
          pltpu.SemaphoreType.DMA,
          pltpu.SemaphoreType.DMA,
          pltpu.SemaphoreType.DMA,
          pltpu.SemaphoreType.DMA,
          pltpu.SemaphoreType.DMA,
      ],
  )


_sc_agg64 = _make_sc_agg(D_HID)
_sc_agg16 = _make_sc_agg(D2)


def _sc_pool_body(keys_hbm, starts_hbm, counts_hbm, h2_hbm, pooled,
                  keys_v, starts_v, counts_v, sel_v, rows_v):
  cid = lax.axis_index("c")
  sid = lax.axis_index("s")
  wid = cid * NS + sid
  pltpu.sync_copy(keys_hbm, keys_v)
  pltpu.sync_copy(starts_hbm, starts_v)
  pltpu.sync_copy(counts_hbm, counts_v)
  iota16 = lax.iota(jnp.int32, 16)

  for j in range((G + NW - 1) // NW):
    g = wid + NW * j

    @pl.when(g < G)
    def _():
      gsplat = jnp.full((16,), g, jnp.int32)
      start = jnp.max(plsc.load_gather(starts_v, [gsplat]))
      cnt = jnp.max(plsc.load_gather(counts_v, [gsplat]))
      for t in range(K // 16):
        sel_v[pl.ds(t * 16, 16)] = jnp.full((16,), SENT, jnp.int32)

      nq = (cnt + 15) // 16

      def pbody(p, _):
        node = start + p
        keyi = plsc.load_gather(keys_v, [jnp.full((16,), node, jnp.int32)])

        def qbody(q, rv):
          kb = q * 16
          kv = keys_v[pl.ds(start + kb, 16)]
          posv = kb + iota16
          valid = posv < cnt
          beats = jnp.logical_or(kv > keyi,
                                 jnp.logical_and(kv == keyi, posv < p))
          return rv + plsc.all_reduce_population_count(
              jnp.logical_and(valid, beats))

        rv = lax.fori_loop(0, nq, qbody, jnp.zeros((16,), jnp.int32))
        rank = jnp.max(rv)
        ranksplat = jnp.full((16,), rank, jnp.int32)
        msk = jnp.logical_and(iota16 == 0, ranksplat < K)
        plsc.store_scatter(sel_v, [ranksplat],
                           jnp.full((16,), node, jnp.int32), mask=msk)
        return 0

      lax.fori_loop(0, cnt, pbody, 0)
      pltpu.sync_copy(h2_hbm.at[sel_v], rows_v)
      pltpu.sync_copy(rows_v, pooled.at[g])


_sc_pool = pl.kernel(
    _sc_pool_body,
    out_type=jax.ShapeDtypeStruct((G, K, D2), jnp.float32),
    mesh=_mesh,
    compiler_params=_sc_params,
    scratch_types=[
        pltpu.VMEM((NPAD,), jnp.float32),
        pltpu.VMEM((GPAD,), jnp.int32),
        pltpu.VMEM((GPAD,), jnp.int32),
        pltpu.VMEM((K,), jnp.int32),
        pltpu.VMEM((K, D2), jnp.float32),
    ],
)


def _tc_a_body(x_ref, w1_ref, deg0_ref, deg1_ref, hp1_ref):
  deg = deg0_ref[:, 0:1] + deg1_ref[:, 0:1] + 1.0
  dinv = lax.rsqrt(deg)
  h = jnp.dot(x_ref[...], w1_ref[...], preferred_element_type=jnp.float32)
  hp1_ref[...] = h * dinv


def _tc_meta_body(batch_ref, starts_ref, counts_ref):
  b = batch_ref[...]
  gid = lax.broadcasted_iota(jnp.int32, (GPAD, NPAD), 0)
  bmat = jnp.broadcast_to(b, (GPAD, NPAD))
  starts_ref[...] = jnp.sum((bmat < gid).astype(jnp.int32), axis=1,
                            keepdims=True)
  counts_ref[...] = jnp.sum((bmat == gid).astype(jnp.int32), axis=1,
                            keepdims=True)


def _tc_b_body(a0_ref, a1_ref, hp1_ref, deg0_ref, deg1_ref, b1_ref, w2_ref,
               hp2_ref):
  deg = deg0_ref[:, 0:1] + deg1_ref[:, 0:1] + 1.0
  dinv = lax.rsqrt(deg)
  h = jnp.tanh((a0_ref[...] + a1_ref[...] + hp1_ref[...]) * dinv + b1_ref[...])
  hp2_ref[...] = jnp.dot(h, w2_ref[...],
                         preferred_element_type=jnp.float32) * dinv


def _tc_c_body(a0_ref, a1_ref, hp2_ref, deg0_ref, deg1_ref, b2_ref,
               h2_ref, key_ref):
  deg = deg0_ref[:, 0:1] + deg1_ref[:, 0:1] + 1.0
  dinv = lax.rsqrt(deg)
  h2 = jnp.tanh((a0_ref[...] + a1_ref[...] + hp2_ref[...]) * dinv + b2_ref[...])
  rows = lax.broadcasted_iota(jnp.int32, (NPAD, 1), 0)
  h2 = jnp.where(rows < N, h2, 0.0)
  h2_ref[...] = h2
  key_ref[...] = h2[:, 1:2]


def _tc_d_body(flat_ref, wl_ref, bl_ref, out_ref):
  out_ref[...] = jnp.dot(flat_ref[...], wl_ref[...],
                         preferred_element_type=jnp.float32) + bl_ref[...]


def _tc_call(body, out_shapes):
  return pl.pallas_call(body, out_shape=out_shapes)


def kernel(x, edge_index, batch, W1, b1, W2, b2, W_lin, b_lin):
  assert x.shape == (N, D_IN) and edge_index.shape == (2, E)
  src = edge_index[0]
  dst = edge_index[1]

  xp = jnp.pad(x, ((0, NPAD - N), (0, 0)))
  batch_p = jnp.concatenate(
      [batch.astype(jnp.int32), jnp.full((NPAD - N,), 1 << 20, jnp.int32)])

  deg0, deg1 = _sc_deg(dst)

  hp1 = _tc_call(_tc_a_body,
                 jax.ShapeDtypeStruct((NPAD, D_HID), jnp.float32))(
                     xp, W1, deg0, deg1)

  starts, counts = _tc_call(
      _tc_meta_body,
      (jax.ShapeDtypeStruct((GPAD, 1), jnp.int32),
       jax.ShapeDtypeStruct((GPAD, 1), jnp.int32)))(batch_p.reshape(1, NPAD))

  a0, a1 = _sc_agg64(src, dst, hp1)

  w2p = jnp.pad(W2, ((0, 0), (0, D2 - W2.shape[1])))
  b2p = jnp.pad(b2, (0, D2 - b2.shape[0]))
  hp2 = _tc_call(_tc_b_body,
                 jax.ShapeDtypeStruct((NPAD, D2), jnp.float32))(
                     a0, a1, hp1, deg0, deg1, b1.reshape(1, D_HID), w2p)

  c0, c1 = _sc_agg16(src, dst, hp2)

  h2, keys = _tc_call(
      _tc_c_body,
      (jax.ShapeDtypeStruct((NPAD, D2), jnp.float32),
       jax.ShapeDtypeStruct((NPAD, 1), jnp.float32)))(
          c0, c1, hp2, deg0, deg1, b2p.reshape(1, D2))

  pooled = _sc_pool(keys.reshape(NPAD), starts.reshape(GPAD),
                    counts.reshape(GPAD), h2)

  wl = jnp.pad(W_lin.reshape(K, 2, 1), ((0, 0), (0, D2 - 2), (0, 0)))
  out = _tc_call(_tc_d_body, jax.ShapeDtypeStruct((G, 1), jnp.float32))(
      pooled.reshape(G, K * D2), wl.reshape(K * D2, 1),
      b_lin.reshape(1, 1))
  return out


# speedup vs baseline: 48.7503x; 1.0207x over previous
import functools

import jax
import jax.numpy as jnp
from jax import lax
from jax.experimental import pallas as pl
from jax.experimental.pallas import tpu as pltpu
from jax.experimental.pallas import tpu_sc as plsc

N = 10000
NPAD = 10240
E = 320000
G = 100
GPAD = 128
K = 64
D_IN = 128
D_HID = 64
D2 = 16
SENT = N
NC, NS = 2, 16
NW = NC * NS
EPW = E // NW
CH = 128
NFULL = EPW // CH
REM = EPW - NFULL * CH
RPS = NPAD // NS
RCH = RPS // CH

_mesh = plsc.VectorSubcoreMesh(core_axis_name="c", subcore_axis_name="s",
                               num_cores=NC, num_subcores=NS)
_sc_params = pltpu.CompilerParams(use_tc_tiling_on_sc=False,
                                  needs_layout_passes=False)


def _fill_rows(ref, rows, vec):
  cols = ref.shape[1] // 16

  @pl.loop(0, rows)
  def _(i):
    for c in range(cols):
      ref[i, pl.ds(c * 16, 16)] = vec


def _zero_slice_loop(sid, buf_v, sh_ref):
  @pl.loop(0, RCH)
  def _(j):
    pltpu.sync_copy(buf_v, sh_ref.at[pl.ds(sid * RPS + j * CH, CH)])


def _copy_out_loop(cid, sid, buf_v, sh_ref, out0, out1):
  @pl.loop(0, RCH)
  def _(j):
    off = sid * RPS + j * CH
    pltpu.sync_copy(sh_ref.at[pl.ds(off, CH)], buf_v)

    @pl.when(cid == 0)
    def _():
      pltpu.sync_copy(buf_v, out0.at[pl.ds(off, CH)])

    @pl.when(cid == 1)
    def _():
      pltpu.sync_copy(buf_v, out1.at[pl.ds(off, CH)])


RING = 6
assert NFULL % RING == 0
NROUND = NFULL // RING


def _repack_idx(idx1_v, idx2_v):
  @pl.loop(0, (NFULL * CH) // 16)
  def _(t):
    idx2_v[t // (CH // 16), pl.ds((t % (CH // 16)) * 16, 16)] = (
        idx1_v[pl.ds(t * 16, 16)])


def _sc_deg_body(dst_hbm, deg0, deg1, didx1_v, didx2_v, idxr_v, ones_v,
                 buf_v, deg_sh, ssem):
  cid = lax.axis_index("c")
  sid = lax.axis_index("s")
  wid = cid * NS + sid
  _fill_rows(ones_v, CH, jnp.full((16,), 1.0, jnp.float32))
  _fill_rows(buf_v, CH, jnp.zeros((16,), jnp.float32))
  _zero_slice_loop(sid, buf_v, deg_sh)

  base = wid * EPW
  pltpu.sync_copy(dst_hbm.at[pl.ds(base, EPW)], didx1_v)
  _repack_idx(didx1_v, didx2_v)
  if REM:
    idxr_v[...] = didx1_v[pl.ds(NFULL * CH, REM)]
  plsc.subcore_barrier()

  @pl.loop(0, NROUND)
  def _(o):
    ibase = o * RING
    for r in range(RING):
      pltpu.async_copy(ones_v, deg_sh.at[didx2_v.at[ibase + r]], ssem,
                       add=True)
    for r in range(RING):
      pltpu.make_async_copy(ones_v, deg_sh.at[didx2_v.at[ibase + r]],
                            ssem).wait()

  if REM:
    pltpu.sync_copy(ones_v.at[pl.ds(0, REM)], deg_sh.at[idxr_v], add=True)

  plsc.subcore_barrier()
  _copy_out_loop(cid, sid, buf_v, deg_sh, deg0, deg1)


_sc_deg = pl.kernel(
    _sc_deg_body,
    out_type=(jax.ShapeDtypeStruct((NPAD, 16), jnp.float32),
              jax.ShapeDtypeStruct((NPAD, 16), jnp.float32)),
    mesh=_mesh,
    compiler_params=_sc_params,
    scratch_types=[
        pltpu.VMEM((EPW,), jnp.int32),
        pltpu.VMEM((NFULL, CH), jnp.int32),
        pltpu.VMEM((REM,), jnp.int32),
        pltpu.VMEM((CH, 16), jnp.float32),
        pltpu.VMEM((CH, 16), jnp.float32),
        pltpu.VMEM_SHARED((NPAD, 16), jnp.float32),
        pltpu.SemaphoreType.DMA,
    ],
)


def _sc_agg_body(src_hbm, dst_hbm, h_hbm, out0, out1,
                 sidx_v, didx1_v, didx2_v, didxr_v, rows_v, buf_v, agg_sh,
                 gs0, gs1, gs2, gs3, gs4, gs5, ssem):
  cid = lax.axis_index("c")
  sid = lax.axis_index("s")
  wid = cid * NS + sid
  gsems = (gs0, gs1, gs2, gs3, gs4, gs5)
  _fill_rows(buf_v, CH, jnp.zeros((16,), jnp.float32))
  _zero_slice_loop(sid, buf_v, agg_sh)

  base = wid * EPW
  pltpu.sync_copy(src_hbm.at[pl.ds(base, EPW)], sidx_v)
  pltpu.sync_copy(dst_hbm.at[pl.ds(base, EPW)], didx1_v)
  _repack_idx(didx1_v, didx2_v)
  if REM:
    didxr_v[...] = didx1_v[pl.ds(NFULL * CH, REM)]
  plsc.subcore_barrier()

  def g_start(i, r):
    pltpu.async_copy(h_hbm.at[sidx_v.at[pl.ds(i * CH, CH)]], rows_v.at[r],
                     gsems[r])

  def g_wait(i, r):
    pltpu.make_async_copy(h_hbm.at[sidx_v.at[pl.ds(i * CH, CH)]],
                          rows_v.at[r], gsems[r]).wait()

  for r in range(RING):
    g_start(r, r)

  @pl.loop(0, NROUND)
  def _(o):
    ibase = o * RING
    for r in range(RING):
      g_wait(ibase + r, r)
      pltpu.async_copy(rows_v.at[r], agg_sh.at[didx2_v.at[ibase + r]], ssem,
                       add=True)
    for r in range(RING):
      pltpu.make_async_copy(rows_v.at[r], agg_sh.at[didx2_v.at[ibase + r]],
                            ssem).wait()
    for r in range(RING):
      nxt = ibase + RING + r

      @pl.when(nxt < NFULL)
      def _():
        g_start(nxt, r)

  if REM:
    off = NFULL * CH
    pltpu.sync_copy(h_hbm.at[sidx_v.at[pl.ds(off, REM)]],
                    rows_v.at[0, pl.ds(0, REM)])
    pltpu.sync_copy(rows_v.at[0, pl.ds(0, REM)], agg_sh.at[didxr_v],
                    add=True)

  plsc.subcore_barrier()
  _copy_out_loop(cid, sid, buf_v, agg_sh, out0, out1)


def _make_sc_agg(width):
  return pl.kernel(
      _sc_agg_body,
      out_type=(jax.ShapeDtypeStruct((NPAD, width), jnp.float32),
                jax.ShapeDtypeStruct((NPAD, width), jnp.float32)),
      mesh=_mesh,
      compiler_params=_sc_params,
      scratch_types=[
          pltpu.VMEM((EPW,), jnp.int32),
          pltpu.VMEM((EPW,), jnp.int32),
          pltpu.VMEM((NFULL, CH), jnp.int32),
          pltpu.VMEM((REM,), jnp.int32),
          pltpu.VMEM((RING, CH, width), jnp.float32),
          pltpu.VMEM((CH, width), jnp.float32),
          pltpu.VMEM_SHARED((NPAD, width), jnp.float32),
          pltpu.SemaphoreType.DMA,
          pltpu.SemaphoreType.DMA,
          pltpu.SemaphoreType.DMA,
          pltpu.SemaphoreType.DMA,
          pltpu.SemaphoreType.DMA,
          pltpu.SemaphoreType.DMA,
          pltpu.SemaphoreType.DMA,
      ],
  )


_sc_agg64 = _make_sc_agg(D_HID)
_sc_agg16 = _make_sc_agg(D2)


def _sc_pool_body(keys_hbm, starts_hbm, counts_hbm, h2_hbm, wl_hbm, outg,
                  keys_v, starts_v, counts_v, sel_v, rows_v, wl_v, row_v):
  cid = lax.axis_index("c")
  sid = lax.axis_index("s")
  wid = cid * NS + sid
  pltpu.sync_copy(keys_hbm, keys_v)
  pltpu.sync_copy(starts_hbm, starts_v)
  pltpu.sync_copy(counts_hbm, counts_v)
  pltpu.sync_copy(wl_hbm, wl_v)
  iota16 = lax.iota(jnp.int32, 16)

  for j in range((G + NW - 1) // NW):
    g = wid + NW * j

    @pl.when(g < G)
    def _():
      gsplat = jnp.full((16,), g, jnp.int32)
      start = jnp.max(plsc.load_gather(starts_v, [gsplat]))
      cnt = jnp.max(plsc.load_gather(counts_v, [gsplat]))
      for t in range(K // 16):
        sel_v[pl.ds(t * 16, 16)] = jnp.full((16,), SENT, jnp.int32)

      nq = (cnt + 15) // 16

      def pbody(p, _):
        node = start + p
        keyi = plsc.load_gather(keys_v, [jnp.full((16,), node, jnp.int32)])

        def qbody(q, rv):
          kb = q * 16
          kv = keys_v[pl.ds(start + kb, 16)]
          posv = kb + iota16
          valid = posv < cnt
          beats = jnp.logical_or(kv > keyi,
                                 jnp.logical_and(kv == keyi, posv < p))
          return rv + plsc.all_reduce_population_count(
              jnp.logical_and(valid, beats))

        rv = lax.fori_loop(0, nq, qbody, jnp.zeros((16,), jnp.int32))
        rank = jnp.max(rv)
        ranksplat = jnp.full((16,), rank, jnp.int32)
        msk = jnp.logical_and(iota16 == 0, ranksplat < K)
        plsc.store_scatter(sel_v, [ranksplat],
                           jnp.full((16,), node, jnp.int32), mask=msk)
        return 0

      lax.fori_loop(0, cnt, pbody, 0)
      pltpu.sync_copy(h2_hbm.at[sel_v], rows_v)

      def dot_body(k, acc):
        return acc + rows_v[k, :] * wl_v[k, :]

      acc = lax.fori_loop(0, K, dot_body, jnp.zeros((16,), jnp.float32))
      tot = jnp.sum(acc)
      row_v[...] = jnp.full((16,), tot, jnp.float32)
      pltpu.sync_copy(row_v, outg.at[g])


_sc_pool = pl.kernel(
    _sc_pool_body,
    out_type=jax.ShapeDtypeStruct((G, 16), jnp.float32),
    mesh=_mesh,
    compiler_params=_sc_params,
    scratch_types=[
        pltpu.VMEM((NPAD,), jnp.float32),
        pltpu.VMEM((GPAD,), jnp.int32),
        pltpu.VMEM((GPAD,), jnp.int32),
        pltpu.VMEM((K,), jnp.int32),
        pltpu.VMEM((K, D2), jnp.float32),
        pltpu.VMEM((K, D2), jnp.float32),
        pltpu.VMEM((16,), jnp.float32),
    ],
)


def _tc_a_body(x_ref, w1_ref, deg0_ref, deg1_ref, batch_ref,
               hp1_ref, starts_ref, counts_ref):
  deg = deg0_ref[:, 0:1] + deg1_ref[:, 0:1] + 1.0
  dinv = lax.rsqrt(deg)
  h = jnp.dot(x_ref[...], w1_ref[...], preferred_element_type=jnp.float32)
  hp1_ref[...] = h * dinv
  b = batch_ref[...]
  gid = lax.broadcasted_iota(jnp.int32, (GPAD, NPAD), 0)
  bmat = jnp.broadcast_to(b, (GPAD, NPAD))
  starts_ref[...] = jnp.sum((bmat < gid).astype(jnp.int32), axis=1,
                            keepdims=True)
  counts_ref[...] = jnp.sum((bmat == gid).astype(jnp.int32), axis=1,
                            keepdims=True)


def _tc_b_body(a0_ref, a1_ref, hp1_ref, deg0_ref, deg1_ref, b1_ref, w2_ref,
               hp2_ref):
  deg = deg0_ref[:, 0:1] + deg1_ref[:, 0:1] + 1.0
  dinv = lax.rsqrt(deg)
  h = jnp.tanh((a0_ref[...] + a1_ref[...] + hp1_ref[...]) * dinv + b1_ref[...])
  hp2_ref[...] = jnp.dot(h, w2_ref[...],
                         preferred_element_type=jnp.float32) * dinv


def _tc_c_body(a0_ref, a1_ref, hp2_ref, deg0_ref, deg1_ref, b2_ref,
               h2_ref, key_ref):
  deg = deg0_ref[:, 0:1] + deg1_ref[:, 0:1] + 1.0
  dinv = lax.rsqrt(deg)
  h2 = jnp.tanh((a0_ref[...] + a1_ref[...] + hp2_ref[...]) * dinv + b2_ref[...])
  rows = lax.broadcasted_iota(jnp.int32, (NPAD, 1), 0)
  h2 = jnp.where(rows < N, h2, 0.0)
  h2_ref[...] = h2
  key_ref[...] = h2[:, 1:2]


def _tc_call(body, out_shapes):
  return pl.pallas_call(body, out_shape=out_shapes)


def kernel(x, edge_index, batch, W1, b1, W2, b2, W_lin, b_lin):
  assert x.shape == (N, D_IN) and edge_index.shape == (2, E)
  src = edge_index[0]
  dst = edge_index[1]

  xp = jnp.pad(x, ((0, NPAD - N), (0, 0)))
  batch_p = jnp.concatenate(
      [batch.astype(jnp.int32), jnp.full((NPAD - N,), 1 << 20, jnp.int32)])

  deg0, deg1 = _sc_deg(dst)

  hp1, starts, counts = _tc_call(
      _tc_a_body,
      (jax.ShapeDtypeStruct((NPAD, D_HID), jnp.float32),
       jax.ShapeDtypeStruct((GPAD, 1), jnp.int32),
       jax.ShapeDtypeStruct((GPAD, 1), jnp.int32)))(
           xp, W1, deg0, deg1, batch_p.reshape(1, NPAD))

  a0, a1 = _sc_agg64(src, dst, hp1)

  w2p = jnp.pad(W2, ((0, 0), (0, D2 - W2.shape[1])))
  b2p = jnp.pad(b2, (0, D2 - b2.shape[0]))
  hp2 = _tc_call(_tc_b_body,
                 jax.ShapeDtypeStruct((NPAD, D2), jnp.float32))(
                     a0, a1, hp1, deg0, deg1, b1.reshape(1, D_HID), w2p)

  c0, c1 = _sc_agg16(src, dst, hp2)

  h2, keys = _tc_call(
      _tc_c_body,
      (jax.ShapeDtypeStruct((NPAD, D2), jnp.float32),
       jax.ShapeDtypeStruct((NPAD, 1), jnp.float32)))(
          c0, c1, hp2, deg0, deg1, b2p.reshape(1, D2))

  wl = jnp.pad(W_lin.reshape(K, 2), ((0, 0), (0, D2 - 2)))
  outg = _sc_pool(keys.reshape(NPAD), starts.reshape(GPAD),
                  counts.reshape(GPAD), h2, wl)
  return outg[:, 0:1] + b_lin


# speedup vs baseline: 51.9976x; 1.0666x over previous
import functools

import jax
import jax.numpy as jnp
from jax import lax
from jax.experimental import pallas as pl
from jax.experimental.pallas import tpu as pltpu
from jax.experimental.pallas import tpu_sc as plsc

N = 10000
NPAD = 10240
E = 320000
G = 100
GPAD = 128
K = 64
D_IN = 128
D_HID = 64
D2 = 16
SENT = N
NC, NS = 2, 16
NW = NC * NS
EPW = E // NW
CH = 128
NFULL = EPW // CH
REM = EPW - NFULL * CH
RPS = NPAD // NS
RCH = RPS // CH

_mesh = plsc.VectorSubcoreMesh(core_axis_name="c", subcore_axis_name="s",
                               num_cores=NC, num_subcores=NS)
_sc_params = pltpu.CompilerParams(use_tc_tiling_on_sc=False,
                                  needs_layout_passes=False)


def _fill_rows(ref, rows, vec):
  cols = ref.shape[1] // 16

  @pl.loop(0, rows)
  def _(i):
    for c in range(cols):
      ref[i, pl.ds(c * 16, 16)] = vec


def _zero_slice_loop(sid, buf_v, sh_ref):
  @pl.loop(0, RCH)
  def _(j):
    pltpu.sync_copy(buf_v, sh_ref.at[pl.ds(sid * RPS + j * CH, CH)])


def _copy_out_loop(cid, sid, buf_v, sh_ref, out0, out1):
  @pl.loop(0, RCH)
  def _(j):
    off = sid * RPS + j * CH
    pltpu.sync_copy(sh_ref.at[pl.ds(off, CH)], buf_v)

    @pl.when(cid == 0)
    def _():
      pltpu.sync_copy(buf_v, out0.at[pl.ds(off, CH)])

    @pl.when(cid == 1)
    def _():
      pltpu.sync_copy(buf_v, out1.at[pl.ds(off, CH)])


RING = 6
assert NFULL % RING == 0
NROUND = NFULL // RING


def _repack_idx(idx1_v, idx2_v):
  @pl.loop(0, (NFULL * CH) // 16)
  def _(t):
    idx2_v[t // (CH // 16), pl.ds((t % (CH // 16)) * 16, 16)] = (
        idx1_v[pl.ds(t * 16, 16)])


def _sc_deg_body(ei_hbm, deg0, deg1, didx1_v, didx2_v, idxr_v, ones_v,
                 buf_v, deg_sh, ssem):
  cid = lax.axis_index("c")
  sid = lax.axis_index("s")
  wid = cid * NS + sid
  base = wid * EPW
  stage = pltpu.async_copy(ei_hbm.at[1, pl.ds(base, EPW)], didx1_v, ssem)
  _fill_rows(ones_v, CH, jnp.full((16,), 1.0, jnp.float32))
  _fill_rows(buf_v, CH, jnp.zeros((16,), jnp.float32))
  _zero_slice_loop(sid, buf_v, deg_sh)
  stage.wait()
  _repack_idx(didx1_v, didx2_v)
  if REM:
    idxr_v[...] = didx1_v[pl.ds(NFULL * CH, REM)]
  plsc.subcore_barrier()

  @pl.loop(0, NROUND)
  def _(o):
    ibase = o * RING
    for r in range(RING):
      pltpu.async_copy(ones_v, deg_sh.at[didx2_v.at[ibase + r]], ssem,
                       add=True)
    for r in range(RING):
      pltpu.make_async_copy(ones_v, deg_sh.at[didx2_v.at[ibase + r]],
                            ssem).wait()

  if REM:
    pltpu.sync_copy(ones_v.at[pl.ds(0, REM)], deg_sh.at[idxr_v], add=True)

  plsc.subcore_barrier()
  _copy_out_loop(cid, sid, buf_v, deg_sh, deg0, deg1)


_sc_deg = pl.kernel(
    _sc_deg_body,
    out_type=(jax.ShapeDtypeStruct((NPAD, 16), jnp.float32),
              jax.ShapeDtypeStruct((NPAD, 16), jnp.float32)),
    mesh=_mesh,
    compiler_params=_sc_params,
    scratch_types=[
        pltpu.VMEM((EPW,), jnp.int32),
        pltpu.VMEM((NFULL, CH), jnp.int32),
        pltpu.VMEM((REM,), jnp.int32),
        pltpu.VMEM((CH, 16), jnp.float32),
        pltpu.VMEM((CH, 16), jnp.float32),
        pltpu.VMEM_SHARED((NPAD, 16), jnp.float32),
        pltpu.SemaphoreType.DMA,
    ],
)


def _sc_agg_body(ei_hbm, h_hbm, out0, out1,
                 sidx_v, didx1_v, didx2_v, didxr_v, rows_v, buf_v, agg_sh,
                 gs0, gs1, gs2, gs3, gs4, gs5, ssem):
  cid = lax.axis_index("c")
  sid = lax.axis_index("s")
  wid = cid * NS + sid
  gsems = (gs0, gs1, gs2, gs3, gs4, gs5)
  base = wid * EPW
  st1 = pltpu.async_copy(ei_hbm.at[0, pl.ds(base, EPW)], sidx_v, gs0)
  st2 = pltpu.async_copy(ei_hbm.at[1, pl.ds(base, EPW)], didx1_v, gs1)
  _fill_rows(buf_v, CH, jnp.zeros((16,), jnp.float32))
  _zero_slice_loop(sid, buf_v, agg_sh)
  st1.wait()
  st2.wait()
  _repack_idx(didx1_v, didx2_v)
  if REM:
    didxr_v[...] = didx1_v[pl.ds(NFULL * CH, REM)]
  plsc.subcore_barrier()

  def g_start(i, r):
    pltpu.async_copy(h_hbm.at[sidx_v.at[pl.ds(i * CH, CH)]], rows_v.at[r],
                     gsems[r])

  def g_wait(i, r):
    pltpu.make_async_copy(h_hbm.at[sidx_v.at[pl.ds(i * CH, CH)]],
                          rows_v.at[r], gsems[r]).wait()

  for r in range(RING):
    g_start(r, r)

  @pl.loop(0, NROUND)
  def _(o):
    ibase = o * RING
    for r in range(RING):
      g_wait(ibase + r, r)
      pltpu.async_copy(rows_v.at[r], agg_sh.at[didx2_v.at[ibase + r]], ssem,
                       add=True)
    for r in range(RING):
      pltpu.make_async_copy(rows_v.at[r], agg_sh.at[didx2_v.at[ibase + r]],
                            ssem).wait()
    for r in range(RING):
      nxt = ibase + RING + r

      @pl.when(nxt < NFULL)
      def _():
        g_start(nxt, r)

  if REM:
    off = NFULL * CH
    pltpu.sync_copy(h_hbm.at[sidx_v.at[pl.ds(off, REM)]],
                    rows_v.at[0, pl.ds(0, REM)])
    pltpu.sync_copy(rows_v.at[0, pl.ds(0, REM)], agg_sh.at[didxr_v],
                    add=True)

  plsc.subcore_barrier()
  _copy_out_loop(cid, sid, buf_v, agg_sh, out0, out1)


def _make_sc_agg(width):
  return pl.kernel(
      _sc_agg_body,
      out_type=(jax.ShapeDtypeStruct((NPAD, width), jnp.float32),
                jax.ShapeDtypeStruct((NPAD, width), jnp.float32)),
      mesh=_mesh,
      compiler_params=_sc_params,
      scratch_types=[
          pltpu.VMEM((EPW,), jnp.int32),
          pltpu.VMEM((EPW,), jnp.int32),
          pltpu.VMEM((NFULL, CH), jnp.int32),
          pltpu.VMEM((REM,), jnp.int32),
          pltpu.VMEM((RING, CH, width), jnp.float32),
          pltpu.VMEM((CH, width), jnp.float32),
          pltpu.VMEM_SHARED((NPAD, width), jnp.float32),
          pltpu.SemaphoreType.DMA,
          pltpu.SemaphoreType.DMA,
          pltpu.SemaphoreType.DMA,
          pltpu.SemaphoreType.DMA,
          pltpu.SemaphoreType.DMA,
          pltpu.SemaphoreType.DMA,
          pltpu.SemaphoreType.DMA,
      ],
  )


_sc_agg64 = _make_sc_agg(D_HID)
_sc_agg16 = _make_sc_agg(D2)


def _sc_pool_body(keys_hbm, starts_hbm, counts_hbm, h2_hbm, wl_hbm, outg,
                  keys_v, starts_v, counts_v, sel_v, rows_v, wl_v, row_v):
  cid = lax.axis_index("c")
  sid = lax.axis_index("s")
  wid = cid * NS + sid
  pltpu.sync_copy(keys_hbm, keys_v)
  pltpu.sync_copy(starts_hbm, starts_v)
  pltpu.sync_copy(counts_hbm, counts_v)
  pltpu.sync_copy(wl_hbm, wl_v)
  iota16 = lax.iota(jnp.int32, 16)

  for j in range((G + NW - 1) // NW):
    g = wid + NW * j

    @pl.when(g < G)
    def _():
      gsplat = jnp.full((16,), g, jnp.int32)
      start = jnp.max(plsc.load_gather(starts_v, [gsplat]))
      cnt = jnp.max(plsc.load_gather(counts_v, [gsplat]))
      for t in range(K // 16):
        sel_v[pl.ds(t * 16, 16)] = jnp.full((16,), SENT, jnp.int32)

      nq = (cnt + 15) // 16

      def pbody(p, _):
        node = start + p
        keyi = plsc.load_gather(keys_v, [jnp.full((16,), node, jnp.int32)])

        def qbody(q, rv):
          kb = q * 16
          kv = keys_v[pl.ds(start + kb, 16)]
          posv = kb + iota16
          valid = posv < cnt
          beats = jnp.logical_or(kv > keyi,
                                 jnp.logical_and(kv == keyi, posv < p))
          return rv + plsc.all_reduce_population_count(
              jnp.logical_and(valid, beats))

        rv = lax.fori_loop(0, nq, qbody, jnp.zeros((16,), jnp.int32))
        rank = jnp.max(rv)
        ranksplat = jnp.full((16,), rank, jnp.int32)
        msk = jnp.logical_and(iota16 == 0, ranksplat < K)
        plsc.store_scatter(sel_v, [ranksplat],
                           jnp.full((16,), node, jnp.int32), mask=msk)
        return 0

      lax.fori_loop(0, cnt, pbody, 0)
      pltpu.sync_copy(h2_hbm.at[sel_v], rows_v)

      def dot_body(k, acc):
        return acc + rows_v[k, :] * wl_v[k, :]

      acc = lax.fori_loop(0, K, dot_body, jnp.zeros((16,), jnp.float32))
      tot = jnp.sum(acc)
      row_v[...] = jnp.full((16,), tot, jnp.float32)
      pltpu.sync_copy(row_v, outg.at[g])


_sc_pool = pl.kernel(
    _sc_pool_body,
    out_type=jax.ShapeDtypeStruct((G, 16), jnp.float32),
    mesh=_mesh,
    compiler_params=_sc_params,
    scratch_types=[
        pltpu.VMEM((NPAD,), jnp.float32),
        pltpu.VMEM((GPAD,), jnp.int32),
        pltpu.VMEM((GPAD,), jnp.int32),
        pltpu.VMEM((K,), jnp.int32),
        pltpu.VMEM((K, D2), jnp.float32),
        pltpu.VMEM((K, D2), jnp.float32),
        pltpu.VMEM((16,), jnp.float32),
    ],
)


def _tc_a_body(x_ref, w1_ref, deg0_ref, deg1_ref, batch_ref,
               hp1_ref, starts_ref, counts_ref):
  deg = deg0_ref[:, 0:1] + deg1_ref[:, 0:1] + 1.0
  dinv = lax.rsqrt(deg)
  h = jnp.dot(x_ref[...], w1_ref[...], preferred_element_type=jnp.float32)
  hp1_ref[0:N, :] = h * dinv[0:N]
  hp1_ref[N:NPAD, :] = jnp.zeros((NPAD - N, D_HID), jnp.float32)
  b = batch_ref[...]
  gid = lax.broadcasted_iota(jnp.int32, (GPAD, N), 0)
  bmat = jnp.broadcast_to(b, (GPAD, N))
  starts_ref[...] = jnp.sum((bmat < gid).astype(jnp.int32), axis=1,
                            keepdims=True)
  counts_ref[...] = jnp.sum((bmat == gid).astype(jnp.int32), axis=1,
                            keepdims=True)


def _tc_b_body(a0_ref, a1_ref, hp1_ref, deg0_ref, deg1_ref, b1_ref, w2_ref,
               hp2_ref):
  deg = deg0_ref[:, 0:1] + deg1_ref[:, 0:1] + 1.0
  dinv = lax.rsqrt(deg)
  h = jnp.tanh((a0_ref[...] + a1_ref[...] + hp1_ref[...]) * dinv + b1_ref[...])
  hp2_ref[...] = jnp.dot(h, w2_ref[...],
                         preferred_element_type=jnp.float32) * dinv


def _tc_c_body(a0_ref, a1_ref, hp2_ref, deg0_ref, deg1_ref, b2_ref,
               h2_ref, key_ref):
  deg = deg0_ref[:, 0:1] + deg1_ref[:, 0:1] + 1.0
  dinv = lax.rsqrt(deg)
  h2 = jnp.tanh((a0_ref[...] + a1_ref[...] + hp2_ref[...]) * dinv + b2_ref[...])
  rows = lax.broadcasted_iota(jnp.int32, (NPAD, 1), 0)
  h2 = jnp.where(rows < N, h2, 0.0)
  h2_ref[...] = h2
  key_ref[...] = h2[:, 1:2]


def _tc_call(body, out_shapes):
  return pl.pallas_call(body, out_shape=out_shapes)


def kernel(x, edge_index, batch, W1, b1, W2, b2, W_lin, b_lin):
  assert x.shape == (N, D_IN) and edge_index.shape == (2, E)

  deg0, deg1 = _sc_deg(edge_index)

  hp1, starts, counts = _tc_call(
      _tc_a_body,
      (jax.ShapeDtypeStruct((NPAD, D_HID), jnp.float32),
       jax.ShapeDtypeStruct((GPAD, 1), jnp.int32),
       jax.ShapeDtypeStruct((GPAD, 1), jnp.int32)))(
           x, W1, deg0, deg1, batch.reshape(1, N))

  a0, a1 = _sc_agg64(edge_index, hp1)

  w2p = jnp.pad(W2, ((0, 0), (0, D2 - W2.shape[1])))
  b2p = jnp.pad(b2, (0, D2 - b2.shape[0]))
  hp2 = _tc_call(_tc_b_body,
                 jax.ShapeDtypeStruct((NPAD, D2), jnp.float32))(
                     a0, a1, hp1, deg0, deg1, b1.reshape(1, D_HID), w2p)

  c0, c1 = _sc_agg16(edge_index, hp2)

  h2, keys = _tc_call(
      _tc_c_body,
      (jax.ShapeDtypeStruct((NPAD, D2), jnp.float32),
       jax.ShapeDtypeStruct((NPAD, 1), jnp.float32)))(
          c0, c1, hp2, deg0, deg1, b2p.reshape(1, D2))

  wl = jnp.pad(W_lin.reshape(K, 2), ((0, 0), (0, D2 - 2)))
  outg = _sc_pool(keys.reshape(NPAD), starts.reshape(GPAD),
                  counts.reshape(GPAD), h2, wl)
  return outg[:, 0:1] + b_lin
